# baseline (speedup 1.0000x reference)
import functools

import jax
import jax.numpy as jnp
from jax import lax
from jax.experimental import pallas as pl
from jax.experimental.pallas import tpu as pltpu
from jax.experimental.pallas import tpu_sc as plsc

VOCAB = 100000
EMBED = 128
BATCH = 1024
SEQ = 512
TYPE_VOCAB = 2

L = 16
NW = 32
N = BATCH * SEQ
ROWS_PER_W = N // NW
CHUNK = 128
NCHUNK = ROWS_PER_W // CHUNK
NSLOT = 3
POS_PERIOD = SEQ // CHUNK
NVEC = EMBED // L
TT_SHIFT = 17
ID_MASK = (1 << TT_SHIFT) - 1


def _body(pids_hbm, word_hbm, pos_hbm, ttab_hbm, out_hbm,
          pidx_v, idx_v, pos_v, ttab_v, wbuf_v, gsems, ssems):
    wid = lax.axis_index("s") * 2 + lax.axis_index("c")
    wbase = wid * ROWS_PER_W

    pltpu.sync_copy(pos_hbm, pos_v)
    pltpu.sync_copy(ttab_hbm, ttab_v)

    @plsc.parallel_loop(0, SEQ)
    def _fold(r):
        for j in range(NVEC):
            sl = pl.ds(j * L, L)
            pos_v[r, sl] = pos_v[r, sl] + ttab_v[0, sl]

    dt = [ttab_v[1, pl.ds(j * L, L)] - ttab_v[0, pl.ds(j * L, L)]
          for j in range(NVEC)]

    def issue(c):
        slot = c % NSLOT
        base = wbase + c * CHUNK
        pltpu.sync_copy(pids_hbm.at[pl.ds(base, CHUNK)], pidx_v.at[slot])
        for j in range(CHUNK // L):
            sl = pl.ds(j * L, L)
            idx_v[slot, sl] = pidx_v[slot, sl] & ID_MASK
        pltpu.async_copy(word_hbm.at[idx_v.at[slot]], wbuf_v.at[slot],
                         gsems.at[slot])

    def wait_gather(c):
        slot = c % NSLOT
        pltpu.make_async_copy(word_hbm.at[idx_v.at[slot]], wbuf_v.at[slot],
                              gsems.at[slot]).wait()

    def compute(c):
        slot = c % NSLOT
        pos_off = (c % POS_PERIOD) * CHUNK

        @plsc.parallel_loop(0, CHUNK // L, unroll=3)
        def _group(g):
            r0 = g * L
            fvec = (pidx_v[slot, pl.ds(r0, L)]
                    >> TT_SHIFT).astype(jnp.float32)
            for k in range(L):
                r = r0 + k
                f = fvec[k]
                prow = pos_off + r
                for j in range(NVEC):
                    sl = pl.ds(j * L, L)
                    plsc.addupdate(wbuf_v.at[slot, r, sl],
                                   pos_v[prow, sl] + f * dt[j])

    def store(c):
        slot = c % NSLOT
        base = wbase + c * CHUNK
        pltpu.async_copy(wbuf_v.at[slot], out_hbm.at[pl.ds(base, CHUNK)],
                         ssems.at[slot])

    def wait_store(c):
        slot = c % NSLOT
        base = wbase + c * CHUNK
        pltpu.make_async_copy(wbuf_v.at[slot], out_hbm.at[pl.ds(base, CHUNK)],
                              ssems.at[slot]).wait()

    issue(0)
    issue(1)

    def step(c, _):
        wait_gather(c)
        compute(c)
        store(c)

        @pl.when(c >= 1)
        def _w():
            wait_store(c - 1)

        @pl.when(c + 2 < NCHUNK)
        def _i():
            issue(c + 2)
        return _

    lax.fori_loop(0, NCHUNK, step, 0, unroll=False)
    wait_store(NCHUNK - 1)


def kernel(input_ids, token_type_ids, word_emb, pos_emb, tok_type_emb):
    ids = input_ids.reshape(N).astype(jnp.int32)
    tt = token_type_ids.reshape(N).astype(jnp.int32)
    packed = ids | (tt << TT_SHIFT)

    mesh = plsc.VectorSubcoreMesh(core_axis_name="c", subcore_axis_name="s")
    out = pl.kernel(
        _body,
        mesh=mesh,
        out_type=jax.ShapeDtypeStruct((N, EMBED), jnp.float32),
        scratch_types=[
            pltpu.VMEM((NSLOT, CHUNK), jnp.int32),
            pltpu.VMEM((NSLOT, CHUNK), jnp.int32),
            pltpu.VMEM((SEQ, EMBED), jnp.float32),
            pltpu.VMEM((TYPE_VOCAB, EMBED), jnp.float32),
            pltpu.VMEM((NSLOT, CHUNK, EMBED), jnp.float32),
            pltpu.SemaphoreType.DMA((NSLOT,)),
            pltpu.SemaphoreType.DMA((NSLOT,)),
        ],
    )(packed, word_emb, pos_emb, tok_type_emb)
    return out.reshape(BATCH, SEQ, EMBED)


# speedup vs baseline: 2.2760x; 2.2760x over previous
import functools

import jax
import jax.numpy as jnp
from jax import lax
from jax.experimental import pallas as pl
from jax.experimental.pallas import tpu as pltpu
from jax.experimental.pallas import tpu_sc as plsc

VOCAB = 100000
EMBED = 128
BATCH = 1024
SEQ = 512
TYPE_VOCAB = 2

L = 16
NW = 32
N = BATCH * SEQ
ROWS_PER_W = N // NW
CHUNK = 128
NCHUNK = ROWS_PER_W // CHUNK
NSLOT = 3
POS_PERIOD = SEQ // CHUNK
NVEC = EMBED // L
TT_SHIFT = 17
ID_MASK = (1 << TT_SHIFT) - 1


def _body(pids_hbm, word_hbm, pos_hbm, ttab_hbm, out_hbm,
          pidx_v, idx_v, pos_v, ttab_v, wbuf_v, gsems, ssems):
    wid = lax.axis_index("s") * 2 + lax.axis_index("c")
    wbase = wid * ROWS_PER_W

    pltpu.sync_copy(pos_hbm, pos_v)
    pltpu.sync_copy(ttab_hbm, ttab_v)

    @plsc.parallel_loop(0, SEQ)
    def _fold(r):
        for j in range(NVEC):
            sl = pl.ds(j * L, L)
            pos_v[r, sl] = pos_v[r, sl] + ttab_v[0, sl]

    dt = [ttab_v[1, pl.ds(j * L, L)] - ttab_v[0, pl.ds(j * L, L)]
          for j in range(NVEC)]

    def issue(c):
        slot = c % NSLOT
        base = wbase + c * CHUNK
        pltpu.sync_copy(pids_hbm.at[pl.ds(base, CHUNK)], pidx_v.at[slot])
        for j in range(CHUNK // L):
            sl = pl.ds(j * L, L)
            idx_v[slot, sl] = pidx_v[slot, sl] & ID_MASK
        pltpu.async_copy(word_hbm.at[idx_v.at[slot]], wbuf_v.at[slot],
                         gsems.at[slot])

    def wait_gather(c):
        slot = c % NSLOT
        pltpu.make_async_copy(word_hbm.at[idx_v.at[slot]], wbuf_v.at[slot],
                              gsems.at[slot]).wait()

    def compute(c):
        slot = c % NSLOT
        pos_off = (c % POS_PERIOD) * CHUNK

        @plsc.parallel_loop(0, CHUNK // L, unroll=2)
        def _group(g):
            r0 = g * L
            fvec = (pidx_v[slot, pl.ds(r0, L)]
                    >> TT_SHIFT).astype(jnp.float32)
            fs = [fvec[k] for k in range(L)]
            for j in range(NVEC):
                sl = pl.ds(j * L, L)
                for k in range(L):
                    r = r0 + k
                    plsc.addupdate(wbuf_v.at[slot, r, sl],
                                   pos_v[pos_off + r, sl] + fs[k] * dt[j])

    def store(c):
        slot = c % NSLOT
        base = wbase + c * CHUNK
        pltpu.async_copy(wbuf_v.at[slot], out_hbm.at[pl.ds(base, CHUNK)],
                         ssems.at[slot])

    def wait_store(c):
        slot = c % NSLOT
        base = wbase + c * CHUNK
        pltpu.make_async_copy(wbuf_v.at[slot], out_hbm.at[pl.ds(base, CHUNK)],
                              ssems.at[slot]).wait()

    issue(0)
    issue(1)

    def step(c, _):
        wait_gather(c)
        compute(c)
        store(c)

        @pl.when(c >= 1)
        def _w():
            wait_store(c - 1)

        @pl.when(c + 2 < NCHUNK)
        def _i():
            issue(c + 2)
        return _

    lax.fori_loop(0, NCHUNK, step, 0, unroll=False)
    wait_store(NCHUNK - 1)


def kernel(input_ids, token_type_ids, word_emb, pos_emb, tok_type_emb):
    ids = input_ids.reshape(N).astype(jnp.int32)
    tt = token_type_ids.reshape(N).astype(jnp.int32)
    packed = ids | (tt << TT_SHIFT)

    mesh = plsc.VectorSubcoreMesh(core_axis_name="c", subcore_axis_name="s")
    out = pl.kernel(
        _body,
        mesh=mesh,
        out_type=jax.ShapeDtypeStruct((N, EMBED), jnp.float32),
        scratch_types=[
            pltpu.VMEM((NSLOT, CHUNK), jnp.int32),
            pltpu.VMEM((NSLOT, CHUNK), jnp.int32),
            pltpu.VMEM((SEQ, EMBED), jnp.float32),
            pltpu.VMEM((TYPE_VOCAB, EMBED), jnp.float32),
            pltpu.VMEM((NSLOT, CHUNK, EMBED), jnp.float32),
            pltpu.SemaphoreType.DMA((NSLOT,)),
            pltpu.SemaphoreType.DMA((NSLOT,)),
        ],
    )(packed, word_emb, pos_emb, tok_type_emb)
    return out.reshape(BATCH, SEQ, EMBED)


# speedup vs baseline: 2.3612x; 1.0375x over previous
import functools

import jax
import jax.numpy as jnp
from jax import lax
from jax.experimental import pallas as pl
from jax.experimental.pallas import tpu as pltpu
from jax.experimental.pallas import tpu_sc as plsc

VOCAB = 100000
EMBED = 128
BATCH = 1024
SEQ = 512
TYPE_VOCAB = 2

L = 16
NW = 32
N = BATCH * SEQ
ROWS_PER_W = N // NW
CHUNK = 128
NCHUNK = ROWS_PER_W // CHUNK
NSLOT = 3
POS_PERIOD = SEQ // CHUNK
NVEC = EMBED // L
HSTREAM = 96
TT_SHIFT = 17
ID_MASK = (1 << TT_SHIFT) - 1


def _body(pids_hbm, word_hbm, pos_hbm, ttab_hbm, out_hbm,
          pidx_v, idx_v, ramp_v, pos_v, ttab_v, wbuf_v, gsems, asems, ssems):
    wid = lax.axis_index("s") * 2 + lax.axis_index("c")
    wbase = wid * ROWS_PER_W

    pltpu.sync_copy(pos_hbm, pos_v)
    pltpu.sync_copy(ttab_hbm, ttab_v)
    for j in range(SEQ // L):
        ramp_v[pl.ds(j * L, L)] = lax.iota(jnp.int32, L) + (j * L)

    @plsc.parallel_loop(0, SEQ)
    def _fold(r):
        for j in range(NVEC):
            sl = pl.ds(j * L, L)
            pos_v[r, sl] = pos_v[r, sl] + ttab_v[0, sl]

    t0 = [ttab_v[0, pl.ds(j * L, L)] for j in range(NVEC)]
    dt = [ttab_v[1, pl.ds(j * L, L)] - t0[j] for j in range(NVEC)]

    def issue(c):
        slot = c % NSLOT
        base = wbase + c * CHUNK
        pltpu.sync_copy(pids_hbm.at[pl.ds(base, CHUNK)], pidx_v.at[slot])
        for j in range(CHUNK // L):
            sl = pl.ds(j * L, L)
            idx_v[slot, sl] = pidx_v[slot, sl] & ID_MASK
        pltpu.async_copy(word_hbm.at[idx_v.at[slot]], wbuf_v.at[slot],
                         gsems.at[slot])

    def wait_gather(c):
        slot = c % NSLOT
        pltpu.make_async_copy(word_hbm.at[idx_v.at[slot]], wbuf_v.at[slot],
                              gsems.at[slot]).wait()

    def fire_pos_add(c):
        slot = c % NSLOT
        pos_off = (c % POS_PERIOD) * CHUNK
        pltpu.async_copy(pos_hbm.at[ramp_v.at[pl.ds(pos_off, HSTREAM)]],
                         wbuf_v.at[slot, pl.ds(0, HSTREAM)],
                         asems.at[slot], add=True)

    def wait_pos_add(c):
        slot = c % NSLOT
        pos_off = (c % POS_PERIOD) * CHUNK
        pltpu.make_async_copy(
            pos_hbm.at[ramp_v.at[pl.ds(pos_off, HSTREAM)]],
            wbuf_v.at[slot, pl.ds(0, HSTREAM)], asems.at[slot]).wait()

    def compute_tail(c):
        slot = c % NSLOT
        pos_off = (c % POS_PERIOD) * CHUNK

        @plsc.parallel_loop(HSTREAM // L, CHUNK // L, unroll=2)
        def _group(g):
            r0 = g * L
            fvec = (pidx_v[slot, pl.ds(r0, L)]
                    >> TT_SHIFT).astype(jnp.float32)
            for k in range(L):
                r = r0 + k
                f = fvec[k]
                prow = pos_off + r
                for j in range(NVEC):
                    sl = pl.ds(j * L, L)
                    plsc.addupdate(wbuf_v.at[slot, r, sl],
                                   pos_v[prow, sl] + f * dt[j])

    def compute_head(c):
        slot = c % NSLOT

        @plsc.parallel_loop(0, HSTREAM // L, unroll=2)
        def _group(g):
            r0 = g * L
            fvec = (pidx_v[slot, pl.ds(r0, L)]
                    >> TT_SHIFT).astype(jnp.float32)
            for k in range(L):
                r = r0 + k
                f = fvec[k]
                for j in range(NVEC):
                    sl = pl.ds(j * L, L)
                    plsc.addupdate(wbuf_v.at[slot, r, sl],
                                   t0[j] + f * dt[j])

    def store(c):
        slot = c % NSLOT
        base = wbase + c * CHUNK
        pltpu.async_copy(wbuf_v.at[slot], out_hbm.at[pl.ds(base, CHUNK)],
                         ssems.at[slot])

    def wait_store(c):
        slot = c % NSLOT
        base = wbase + c * CHUNK
        pltpu.make_async_copy(wbuf_v.at[slot], out_hbm.at[pl.ds(base, CHUNK)],
                              ssems.at[slot]).wait()

    issue(0)
    issue(1)

    def step(c, _):
        wait_gather(c)
        fire_pos_add(c)
        compute_tail(c)
        wait_pos_add(c)
        compute_head(c)
        store(c)

        @pl.when(c >= 1)
        def _w():
            wait_store(c - 1)

        @pl.when(c + 2 < NCHUNK)
        def _i():
            issue(c + 2)
        return _

    lax.fori_loop(0, NCHUNK, step, 0, unroll=False)
    wait_store(NCHUNK - 1)


def kernel(input_ids, token_type_ids, word_emb, pos_emb, tok_type_emb):
    ids = input_ids.reshape(N).astype(jnp.int32)
    tt = token_type_ids.reshape(N).astype(jnp.int32)
    packed = ids | (tt << TT_SHIFT)

    mesh = plsc.VectorSubcoreMesh(core_axis_name="c", subcore_axis_name="s")
    out = pl.kernel(
        _body,
        mesh=mesh,
        out_type=jax.ShapeDtypeStruct((N, EMBED), jnp.float32),
        scratch_types=[
            pltpu.VMEM((NSLOT, CHUNK), jnp.int32),
            pltpu.VMEM((NSLOT, CHUNK), jnp.int32),
            pltpu.VMEM((SEQ,), jnp.int32),
            pltpu.VMEM((SEQ, EMBED), jnp.float32),
            pltpu.VMEM((TYPE_VOCAB, EMBED), jnp.float32),
            pltpu.VMEM((NSLOT, CHUNK, EMBED), jnp.float32),
            pltpu.SemaphoreType.DMA((NSLOT,)),
            pltpu.SemaphoreType.DMA((NSLOT,)),
            pltpu.SemaphoreType.DMA((NSLOT,)),
        ],
    )(packed, word_emb, pos_emb, tok_type_emb)
    return out.reshape(BATCH, SEQ, EMBED)


# speedup vs baseline: 2.5807x; 1.0930x over previous
import functools

import jax
import jax.numpy as jnp
from jax import lax
from jax.experimental import pallas as pl
from jax.experimental.pallas import tpu as pltpu
from jax.experimental.pallas import tpu_sc as plsc

VOCAB = 100000
EMBED = 128
BATCH = 1024
SEQ = 512
TYPE_VOCAB = 2

L = 16
NW = 32
N = BATCH * SEQ
ROWS_PER_W = N // NW
CHUNK = 128
NCHUNK = ROWS_PER_W // CHUNK
NSLOT = 3
POS_PERIOD = SEQ // CHUNK
NVEC = EMBED // L
HSTREAM = 64
TT_SHIFT = 17
ID_MASK = (1 << TT_SHIFT) - 1


def _body(pids_hbm, word_hbm, pos_hbm, ttab_hbm, out_hbm,
          pidx_v, idx_v, ramp_v, pos_v, ttab_v, wbuf_v, gsems, asems, ssems):
    wid = lax.axis_index("s") * 2 + lax.axis_index("c")
    wbase = wid * ROWS_PER_W

    pltpu.sync_copy(pos_hbm, pos_v)
    pltpu.sync_copy(ttab_hbm, ttab_v)
    for j in range(SEQ // L):
        ramp_v[pl.ds(j * L, L)] = lax.iota(jnp.int32, L) + (j * L)

    @plsc.parallel_loop(0, SEQ)
    def _fold(r):
        for j in range(NVEC):
            sl = pl.ds(j * L, L)
            pos_v[r, sl] = pos_v[r, sl] + ttab_v[0, sl]

    t0 = [ttab_v[0, pl.ds(j * L, L)] for j in range(NVEC)]
    dt = [ttab_v[1, pl.ds(j * L, L)] - t0[j] for j in range(NVEC)]

    def issue(c):
        slot = c % NSLOT
        base = wbase + c * CHUNK
        pltpu.sync_copy(pids_hbm.at[pl.ds(base, CHUNK)], pidx_v.at[slot])
        for j in range(CHUNK // L):
            sl = pl.ds(j * L, L)
            idx_v[slot, sl] = pidx_v[slot, sl] & ID_MASK
        pltpu.async_copy(word_hbm.at[idx_v.at[slot]], wbuf_v.at[slot],
                         gsems.at[slot])

    def wait_gather(c):
        slot = c % NSLOT
        pltpu.make_async_copy(word_hbm.at[idx_v.at[slot]], wbuf_v.at[slot],
                              gsems.at[slot]).wait()

    def fire_pos_add(c):
        slot = c % NSLOT
        pos_off = (c % POS_PERIOD) * CHUNK
        pltpu.async_copy(pos_hbm.at[ramp_v.at[pl.ds(pos_off, HSTREAM)]],
                         wbuf_v.at[slot, pl.ds(0, HSTREAM)],
                         asems.at[slot], add=True)

    def wait_pos_add(c):
        slot = c % NSLOT
        pos_off = (c % POS_PERIOD) * CHUNK
        pltpu.make_async_copy(
            pos_hbm.at[ramp_v.at[pl.ds(pos_off, HSTREAM)]],
            wbuf_v.at[slot, pl.ds(0, HSTREAM)], asems.at[slot]).wait()

    def compute_tail(c):
        slot = c % NSLOT
        pos_off = (c % POS_PERIOD) * CHUNK

        @plsc.parallel_loop(HSTREAM // L, CHUNK // L, unroll=2)
        def _group(g):
            r0 = g * L
            fvec = (pidx_v[slot, pl.ds(r0, L)]
                    >> TT_SHIFT).astype(jnp.float32)
            for k in range(L):
                r = r0 + k
                f = fvec[k]
                prow = pos_off + r
                for j in range(NVEC):
                    sl = pl.ds(j * L, L)
                    plsc.addupdate(wbuf_v.at[slot, r, sl],
                                   pos_v[prow, sl] + f * dt[j])

    def compute_head(c):
        slot = c % NSLOT

        @plsc.parallel_loop(0, HSTREAM // L, unroll=2)
        def _group(g):
            r0 = g * L
            fvec = (pidx_v[slot, pl.ds(r0, L)]
                    >> TT_SHIFT).astype(jnp.float32)
            for k in range(L):
                r = r0 + k
                f = fvec[k]
                for j in range(NVEC):
                    sl = pl.ds(j * L, L)
                    plsc.addupdate(wbuf_v.at[slot, r, sl],
                                   t0[j] + f * dt[j])

    def store(c):
        slot = c % NSLOT
        base = wbase + c * CHUNK
        pltpu.async_copy(wbuf_v.at[slot], out_hbm.at[pl.ds(base, CHUNK)],
                         ssems.at[slot])

    def wait_store(c):
        slot = c % NSLOT
        base = wbase + c * CHUNK
        pltpu.make_async_copy(wbuf_v.at[slot], out_hbm.at[pl.ds(base, CHUNK)],
                              ssems.at[slot]).wait()

    issue(0)
    issue(1)

    def step(c, _):
        wait_gather(c)
        fire_pos_add(c)
        compute_tail(c)
        wait_pos_add(c)
        compute_head(c)
        store(c)

        @pl.when(c >= 1)
        def _w():
            wait_store(c - 1)

        @pl.when(c + 2 < NCHUNK)
        def _i():
            issue(c + 2)
        return _

    lax.fori_loop(0, NCHUNK, step, 0, unroll=False)
    wait_store(NCHUNK - 1)


def kernel(input_ids, token_type_ids, word_emb, pos_emb, tok_type_emb):
    ids = input_ids.reshape(N).astype(jnp.int32)
    tt = token_type_ids.reshape(N).astype(jnp.int32)
    packed = ids | (tt << TT_SHIFT)

    mesh = plsc.VectorSubcoreMesh(core_axis_name="c", subcore_axis_name="s")
    out = pl.kernel(
        _body,
        mesh=mesh,
        out_type=jax.ShapeDtypeStruct((N, EMBED), jnp.float32),
        scratch_types=[
            pltpu.VMEM((NSLOT, CHUNK), jnp.int32),
            pltpu.VMEM((NSLOT, CHUNK), jnp.int32),
            pltpu.VMEM((SEQ,), jnp.int32),
            pltpu.VMEM((SEQ, EMBED), jnp.float32),
            pltpu.VMEM((TYPE_VOCAB, EMBED), jnp.float32),
            pltpu.VMEM((NSLOT, CHUNK, EMBED), jnp.float32),
            pltpu.SemaphoreType.DMA((NSLOT,)),
            pltpu.SemaphoreType.DMA((NSLOT,)),
            pltpu.SemaphoreType.DMA((NSLOT,)),
        ],
    )(packed, word_emb, pos_emb, tok_type_emb)
    return out.reshape(BATCH, SEQ, EMBED)
